# baseline (speedup 1.0000x reference)
import functools

import jax
import jax.numpy as jnp
from jax import lax
from jax.experimental import pallas as pl
from jax.experimental.pallas import tpu as pltpu
from jax.experimental.pallas import tpu_sc as plsc

B = 16384
D = 64
L = 16
NC = 2
NS = 16
NW = NC * NS
RPW = B // NW
CH = 256
NCHUNK = RPW // CH


def _sc_body(e1_hbm, rel_hbm, e2_hbm, er_hbm, ei_hbm, rr_hbm, ri_hbm,
             out_hbm,
             e1_v, rel_v, e2_v,
             a_r0, a_i0, r_r0, r_i0, b_r0, b_i0,
             s_v, out_v, sem0):
    wid = lax.axis_index("s") * NC + lax.axis_index("c")
    row0 = wid * RPW

    pltpu.sync_copy(e1_hbm.at[pl.ds(row0, RPW)], e1_v)
    pltpu.sync_copy(rel_hbm.at[pl.ds(row0, RPW)], rel_v)
    pltpu.sync_copy(e2_hbm.at[pl.ds(row0, RPW)], e2_v)

    bufs = [
        (a_r0, a_i0, r_r0, r_i0, b_r0, b_i0, sem0),
    ]

    def start(c, buf):
        a_r, a_i, r_r, r_i, b_r, b_i, sem = buf
        sl = pl.ds(c * CH, CH)
        return [
            pltpu.async_copy(er_hbm.at[e1_v.at[sl]], a_r, sem),
            pltpu.async_copy(ei_hbm.at[e1_v.at[sl]], a_i, sem),
            pltpu.async_copy(rr_hbm.at[rel_v.at[sl]], r_r, sem),
            pltpu.async_copy(ri_hbm.at[rel_v.at[sl]], r_i, sem),
            pltpu.async_copy(er_hbm.at[e2_v.at[sl]], b_r, sem),
            pltpu.async_copy(ei_hbm.at[e2_v.at[sl]], b_i, sem),
        ]

    def compute(c, buf):
        a_r, a_i, r_r, r_i, b_r, b_i, _ = buf

        def group_body(g, carry2):
            def row_body(r, carry3):
                row = g * L + r
                acc = jnp.zeros((L,), jnp.float32)
                for k in range(D // L):
                    sl = pl.ds(k * L, L)
                    ar = a_r[row, sl]
                    ai = a_i[row, sl]
                    rr = r_r[row, sl]
                    ri = r_i[row, sl]
                    br = b_r[row, sl]
                    bi = b_i[row, sl]
                    acc = acc + br * (ar * rr - ai * ri) + bi * (ar * ri + ai * rr)
                s_v[pl.ds(pl.multiple_of(r * L, L), L)] = acc
                return carry3

            lax.fori_loop(0, L, row_body, 0)
            lane = lax.iota(jnp.int32, L)
            tot = jnp.zeros((L,), jnp.float32)
            for j in range(L):
                tot = tot + plsc.load_gather(s_v, [lane * L + j])
            res = 1.0 / (1.0 + jnp.exp(-tot))
            off = pl.multiple_of(c * CH + g * L, L)
            out_v[pl.ds(off, L)] = res
            return carry2

        lax.fori_loop(0, CH // L, group_body, 0)

    for c in range(NCHUNK):
        cps = start(c, bufs[0])
        for cp in cps:
            cp.wait()
        compute(c, bufs[0])

    pltpu.sync_copy(out_v, out_hbm.at[pl.ds(row0, RPW)])


@jax.jit
def _scores(e1_idx, rel_idx, e2_idx, ent_real, ent_img, rel_real, rel_img):
    mesh = plsc.VectorSubcoreMesh(core_axis_name="c", subcore_axis_name="s")
    fn = pl.kernel(
        _sc_body,
        mesh=mesh,
        compiler_params=pltpu.CompilerParams(
            needs_layout_passes=False, use_tc_tiling_on_sc=False
        ),
        out_type=jax.ShapeDtypeStruct((B,), jnp.float32),
        scratch_types=(
            [pltpu.VMEM((RPW,), jnp.int32)] * 3
            + [pltpu.VMEM((CH, D), jnp.float32)] * 6
            + [
                pltpu.VMEM((L * L,), jnp.float32),
                pltpu.VMEM((RPW,), jnp.float32),
                pltpu.SemaphoreType.DMA,
            ]
        ),
    )
    return fn(e1_idx, rel_idx, e2_idx, ent_real, ent_img, rel_real, rel_img)


def kernel(e1_idx, rel_idx, e2_idx, ent_real, ent_img, rel_real, rel_img):
    e1 = e1_idx.astype(jnp.int32)
    rel = rel_idx.astype(jnp.int32)
    e2 = e2_idx.astype(jnp.int32)
    out = _scores(e1, rel, e2, ent_real, ent_img, rel_real, rel_img)
    return (out, jnp.float32(0.0))


# speedup vs baseline: 1.0051x; 1.0051x over previous
import functools

import jax
import jax.numpy as jnp
from jax import lax
from jax.experimental import pallas as pl
from jax.experimental.pallas import tpu as pltpu
from jax.experimental.pallas import tpu_sc as plsc

B = 16384
D = 64
L = 16
NC = 2
NS = 16
NW = NC * NS
RPW = B // NW
CH = 128
NCHUNK = RPW // CH


def _sc_body(e1_hbm, rel_hbm, e2_hbm, er_hbm, ei_hbm, rr_hbm, ri_hbm,
             out_hbm,
             e1_v, rel_v, e2_v,
             a_r0, a_i0, r_r0, r_i0, b_r0, b_i0,
             a_r1, a_i1, r_r1, r_i1, b_r1, b_i1,
             s_v, out_v, sem0, sem1):
    wid = lax.axis_index("s") * NC + lax.axis_index("c")
    row0 = wid * RPW

    pltpu.sync_copy(e1_hbm.at[pl.ds(row0, RPW)], e1_v)
    pltpu.sync_copy(rel_hbm.at[pl.ds(row0, RPW)], rel_v)
    pltpu.sync_copy(e2_hbm.at[pl.ds(row0, RPW)], e2_v)

    bufs = [
        (a_r0, a_i0, r_r0, r_i0, b_r0, b_i0, sem0),
        (a_r1, a_i1, r_r1, r_i1, b_r1, b_i1, sem1),
    ]

    def start(c, buf):
        a_r, a_i, r_r, r_i, b_r, b_i, sem = buf
        sl = pl.ds(c * CH, CH)
        return [
            pltpu.async_copy(er_hbm.at[e1_v.at[sl]], a_r, sem),
            pltpu.async_copy(ei_hbm.at[e1_v.at[sl]], a_i, sem),
            pltpu.async_copy(rr_hbm.at[rel_v.at[sl]], r_r, sem),
            pltpu.async_copy(ri_hbm.at[rel_v.at[sl]], r_i, sem),
            pltpu.async_copy(er_hbm.at[e2_v.at[sl]], b_r, sem),
            pltpu.async_copy(ei_hbm.at[e2_v.at[sl]], b_i, sem),
        ]

    def compute(c, buf):
        a_r, a_i, r_r, r_i, b_r, b_i, _ = buf

        def group_body(g, carry2):
            def row_body(r, carry3):
                row = g * L + r
                acc = jnp.zeros((L,), jnp.float32)
                for k in range(D // L):
                    sl = pl.ds(k * L, L)
                    ar = a_r[row, sl]
                    ai = a_i[row, sl]
                    rr = r_r[row, sl]
                    ri = r_i[row, sl]
                    br = b_r[row, sl]
                    bi = b_i[row, sl]
                    acc = acc + br * (ar * rr - ai * ri) + bi * (ar * ri + ai * rr)
                s_v[pl.ds(pl.multiple_of(r * L, L), L)] = acc
                return carry3

            lax.fori_loop(0, L, row_body, 0)
            lane = lax.iota(jnp.int32, L)
            tot = jnp.zeros((L,), jnp.float32)
            for j in range(L):
                tot = tot + plsc.load_gather(s_v, [lane * L + j])
            res = 1.0 / (1.0 + jnp.exp(-tot))
            off = pl.multiple_of(c * CH + g * L, L)
            out_v[pl.ds(off, L)] = res
            return carry2

        lax.fori_loop(0, CH // L, group_body, 0)

    cps = start(0, bufs[0])
    for c in range(NCHUNK):
        nxt = None
        if c + 1 < NCHUNK:
            nxt = start(c + 1, bufs[(c + 1) % 2])
        for cp in cps:
            cp.wait()
        compute(c, bufs[c % 2])
        cps = nxt

    pltpu.sync_copy(out_v, out_hbm.at[pl.ds(row0, RPW)])


@jax.jit
def _scores(e1_idx, rel_idx, e2_idx, ent_real, ent_img, rel_real, rel_img):
    mesh = plsc.VectorSubcoreMesh(core_axis_name="c", subcore_axis_name="s")
    fn = pl.kernel(
        _sc_body,
        mesh=mesh,
        compiler_params=pltpu.CompilerParams(
            needs_layout_passes=False, use_tc_tiling_on_sc=False
        ),
        out_type=jax.ShapeDtypeStruct((B,), jnp.float32),
        scratch_types=(
            [pltpu.VMEM((RPW,), jnp.int32)] * 3
            + [pltpu.VMEM((CH, D), jnp.float32)] * 12
            + [
                pltpu.VMEM((L * L,), jnp.float32),
                pltpu.VMEM((RPW,), jnp.float32),
                pltpu.SemaphoreType.DMA,
                pltpu.SemaphoreType.DMA,
            ]
        ),
    )
    return fn(e1_idx, rel_idx, e2_idx, ent_real, ent_img, rel_real, rel_img)


def kernel(e1_idx, rel_idx, e2_idx, ent_real, ent_img, rel_real, rel_img):
    e1 = e1_idx.astype(jnp.int32)
    rel = rel_idx.astype(jnp.int32)
    e2 = e2_idx.astype(jnp.int32)
    out = _scores(e1, rel, e2, ent_real, ent_img, rel_real, rel_img)
    return (out, jnp.float32(0.0))
